# baseline (speedup 1.0000x reference)
import functools

import jax
import jax.numpy as jnp
from jax import lax
from jax.experimental import pallas as pl
from jax.experimental.pallas import tpu as pltpu
from jax.experimental.pallas import tpu_sc as plsc

_C = 1000
_B = 16384
_NC = 2
_NS = 16
_NW = _NC * _NS
_RPW = _B // _NW
_IROWS = 3 * _RPW // 128


def _sc_dot_body(logits_hbm, y_hbm, out_hbm, yv, idxv, valv, accv, sem):
    cid = lax.axis_index("c")
    sid = lax.axis_index("s")
    wid = sid * _NC + cid
    base = wid * _RPW

    pltpu.sync_copy(y_hbm.at[pl.ds(base, _RPW)], yv)

    lanes = lax.iota(jnp.int32, 16)
    c32 = jnp.int32(_C)
    one = jnp.int32(1)
    for i in range(_RPW // 16):
        yy = yv[pl.ds(i * 16, 16)]
        rowbase = (base + i * 16 + lanes) * c32
        prev = lax.rem(yy - one + c32, c32)
        nxt = lax.rem(yy + one, c32)
        j = i // 8
        col = (i % 8) * 16
        idxv[j, pl.ds(col, 16)] = rowbase + yy
        idxv[4 + j, pl.ds(col, 16)] = rowbase + prev
        idxv[8 + j, pl.ds(col, 16)] = rowbase + nxt

    descs = [
        pltpu.async_copy(logits_hbm.at[idxv.at[j]], valv.at[j], sem)
        for j in range(_IROWS)
    ]
    for d in descs:
        d.wait()

    acc = jnp.zeros((16,), jnp.float32)
    for j in range(_IROWS):
        w = jnp.float32(0.8 if j < 4 else 0.1)
        for k in range(8):
            acc = acc + w * valv[j, pl.ds(k * 16, 16)]
    accv[...] = acc
    pltpu.sync_copy(accv, out_hbm.at[pl.ds(wid * 16, 16)])


_BK = 1024


def _lse_block(logits_ref, out_ref):
    i = pl.program_id(0)
    x = logits_ref[...]
    m = jnp.max(x, axis=1, keepdims=True)
    lse = jnp.log(jnp.sum(jnp.exp(x - m), axis=1)) + m[:, 0]
    partial = jnp.sum(lse).reshape(1, 1)

    @pl.when(i == 0)
    def _():
        out_ref[...] = partial

    @pl.when(i != 0)
    def _():
        out_ref[...] += partial


def kernel(logits, y_true):
    b, c = logits.shape
    y = y_true.astype(jnp.int32)
    flat = logits.reshape(b * c)

    mesh = plsc.VectorSubcoreMesh(
        core_axis_name="c", subcore_axis_name="s", num_cores=_NC, num_subcores=_NS
    )
    sc_dot = pl.kernel(
        _sc_dot_body,
        out_type=jax.ShapeDtypeStruct((_NW * 16,), jnp.float32),
        mesh=mesh,
        scratch_types=[
            pltpu.VMEM((_RPW,), jnp.int32),
            pltpu.VMEM((_IROWS, 128), jnp.int32),
            pltpu.VMEM((_IROWS, 128), jnp.float32),
            pltpu.VMEM((16,), jnp.float32),
            pltpu.SemaphoreType.DMA,
        ],
    )
    dots = sc_dot(flat, y)

    lse_sum = pl.pallas_call(
        _lse_block,
        grid=(b // _BK,),
        in_specs=[pl.BlockSpec((_BK, c), lambda i: (i, 0))],
        out_specs=pl.BlockSpec((1, 1), lambda i: (0, 0)),
        out_shape=jax.ShapeDtypeStruct((1, 1), jnp.float32),
    )(logits)

    return (lse_sum[0, 0] - jnp.sum(dots)) / b
